# Initial kernel scaffold; baseline (speedup 1.0000x reference)
import jax
import jax.numpy as jnp
from jax.experimental import pallas as pl


def kernel(embed, head, rel, tail, w_relations):
    raise NotImplementedError("write your pallas kernel here")


# speedup vs baseline: 2.2295x; 2.2295x over previous
import functools

import jax
import jax.numpy as jnp
from jax import lax
from jax.experimental import pallas as pl
from jax.experimental.pallas import tpu as pltpu
from jax.experimental.pallas import tpu_sc as plsc

NUM_ENTITIES = 100000
NUM_RELS = 1000
FEAT_DIM = 128
BATCH = 16384

NC = 2
NS = 16
L = 16
NW = NC * NS
ROWS_PER_W = BATCH // NW
CHUNK = 128
NCHUNKS = ROWS_PER_W // CHUNK
GROUPS = CHUNK // L
DCH = FEAT_DIM // L


def _sc_body(embed_hbm, head_hbm, rel_hbm, tail_hbm, wrel_hbm, out_hbm,
             hidx_v, ridx_v, tidx_v, h_v, r_v, t_v, out_v, sem):
    wid = lax.axis_index("s") * NC + lax.axis_index("c")
    base = wid * ROWS_PER_W

    pltpu.sync_copy(head_hbm.at[pl.ds(base, ROWS_PER_W)], hidx_v)
    pltpu.sync_copy(rel_hbm.at[pl.ds(base, ROWS_PER_W)], ridx_v)
    pltpu.sync_copy(tail_hbm.at[pl.ds(base, ROWS_PER_W)], tidx_v)

    lane = lax.broadcasted_iota(jnp.int32, (L,), 0)

    for c in range(NCHUNKS):
        cp_h = pltpu.async_copy(
            embed_hbm.at[hidx_v.at[pl.ds(c * CHUNK, CHUNK)]], h_v, sem)
        cp_r = pltpu.async_copy(
            wrel_hbm.at[ridx_v.at[pl.ds(c * CHUNK, CHUNK)]], r_v, sem)
        cp_t = pltpu.async_copy(
            embed_hbm.at[tidx_v.at[pl.ds(c * CHUNK, CHUNK)]], t_v, sem)
        cp_h.wait()
        cp_r.wait()
        cp_t.wait()

        def group_body(g, _):
            rows = g * L + lane

            def dstep(_, carry):
                acc, col = carry
                for _ in range(8):
                    hv = plsc.load_gather(h_v, [rows, col])
                    rv = plsc.load_gather(r_v, [rows, col])
                    tv = plsc.load_gather(t_v, [rows, col])
                    acc = acc + hv * rv * tv
                    col = (col + 1) & (FEAT_DIM - 1)
                return acc, col

            acc, _ = lax.fori_loop(
                0, FEAT_DIM // 8, dstep,
                (jnp.zeros((L,), jnp.float32), lane))
            out_v[pl.ds(c * CHUNK + g * L, L)] = acc
            return 0

        lax.fori_loop(0, GROUPS, group_body, 0)

    pltpu.sync_copy(out_v, out_hbm.at[pl.ds(base, ROWS_PER_W)])


@jax.jit
def _dist_mult(embed, head, rel, tail, w_relations):
    mesh = plsc.VectorSubcoreMesh(core_axis_name="c", subcore_axis_name="s")
    run = pl.kernel(
        _sc_body,
        out_type=jax.ShapeDtypeStruct((BATCH,), jnp.float32),
        mesh=mesh,
        compiler_params=pltpu.CompilerParams(needs_layout_passes=False),
        scratch_types=[
            pltpu.VMEM((ROWS_PER_W,), jnp.int32),
            pltpu.VMEM((ROWS_PER_W,), jnp.int32),
            pltpu.VMEM((ROWS_PER_W,), jnp.int32),
            pltpu.VMEM((CHUNK, FEAT_DIM), jnp.float32),
            pltpu.VMEM((CHUNK, FEAT_DIM), jnp.float32),
            pltpu.VMEM((CHUNK, FEAT_DIM), jnp.float32),
            pltpu.VMEM((ROWS_PER_W,), jnp.float32),
            pltpu.SemaphoreType.DMA,
        ],
    )
    return run(embed, head, rel, tail, w_relations)


def kernel(embed, head, rel, tail, w_relations):
    head = head.astype(jnp.int32)
    rel = rel.astype(jnp.int32)
    tail = tail.astype(jnp.int32)
    return _dist_mult(embed, head, rel, tail, w_relations)


# speedup vs baseline: 2.4986x; 1.1207x over previous
import jax
import jax.numpy as jnp
from jax import lax
from jax.experimental import pallas as pl
from jax.experimental.pallas import tpu as pltpu
from jax.experimental.pallas import tpu_sc as plsc

NUM_ENTITIES = 100000
NUM_RELS = 1000
FEAT_DIM = 128
BATCH = 16384

NC = 2
NS = 16
L = 16
NW = NC * NS
ROWS_PER_W = BATCH // NW
CHUNK = 128
NCHUNKS = ROWS_PER_W // CHUNK
GROUPS = CHUNK // L
UNROLL = 8


def _sc_body(embed_hbm, head_hbm, rel_hbm, tail_hbm, wrel_hbm, out_hbm,
             hidx_v, ridx_v, tidx_v,
             h0, r0, t0, h1, r1, t1, out_v, sem0, sem1):
    wid = lax.axis_index("s") * NC + lax.axis_index("c")
    base = wid * ROWS_PER_W

    pltpu.sync_copy(head_hbm.at[pl.ds(base, ROWS_PER_W)], hidx_v)
    pltpu.sync_copy(rel_hbm.at[pl.ds(base, ROWS_PER_W)], ridx_v)
    pltpu.sync_copy(tail_hbm.at[pl.ds(base, ROWS_PER_W)], tidx_v)

    bufs = [(h0, r0, t0, sem0), (h1, r1, t1, sem1)]
    lane = lax.broadcasted_iota(jnp.int32, (L,), 0)

    def issue(c):
        hb, rb, tb, sem = bufs[c % 2]
        sl = pl.ds(c * CHUNK, CHUNK)
        return [
            pltpu.async_copy(embed_hbm.at[hidx_v.at[sl]], hb, sem),
            pltpu.async_copy(wrel_hbm.at[ridx_v.at[sl]], rb, sem),
            pltpu.async_copy(embed_hbm.at[tidx_v.at[sl]], tb, sem),
        ]

    pending = {0: issue(0)}
    for c in range(NCHUNKS):
        if c + 1 < NCHUNKS:
            pending[c + 1] = issue(c + 1)
        for cp in pending.pop(c):
            cp.wait()
        h_v, r_v, t_v, _ = bufs[c % 2]

        def group_body(g, _):
            rows = g * L + lane

            def dstep(_, carry):
                acc, col = carry
                for _ in range(UNROLL):
                    hv = plsc.load_gather(h_v, [rows, col])
                    rv = plsc.load_gather(r_v, [rows, col])
                    tv = plsc.load_gather(t_v, [rows, col])
                    acc = acc + hv * rv * tv
                    col = (col + 1) & (FEAT_DIM - 1)
                return acc, col

            acc, _ = lax.fori_loop(
                0, FEAT_DIM // UNROLL, dstep,
                (jnp.zeros((L,), jnp.float32), lane))
            out_v[pl.ds(c * CHUNK + g * L, L)] = acc
            return 0

        lax.fori_loop(0, GROUPS, group_body, 0)

    pltpu.sync_copy(out_v, out_hbm.at[pl.ds(base, ROWS_PER_W)])


@jax.jit
def _dist_mult(embed, head, rel, tail, w_relations):
    mesh = plsc.VectorSubcoreMesh(core_axis_name="c", subcore_axis_name="s")
    run = pl.kernel(
        _sc_body,
        out_type=jax.ShapeDtypeStruct((BATCH,), jnp.float32),
        mesh=mesh,
        compiler_params=pltpu.CompilerParams(needs_layout_passes=False),
        scratch_types=[
            pltpu.VMEM((ROWS_PER_W,), jnp.int32),
            pltpu.VMEM((ROWS_PER_W,), jnp.int32),
            pltpu.VMEM((ROWS_PER_W,), jnp.int32),
            pltpu.VMEM((CHUNK, FEAT_DIM), jnp.float32),
            pltpu.VMEM((CHUNK, FEAT_DIM), jnp.float32),
            pltpu.VMEM((CHUNK, FEAT_DIM), jnp.float32),
            pltpu.VMEM((CHUNK, FEAT_DIM), jnp.float32),
            pltpu.VMEM((CHUNK, FEAT_DIM), jnp.float32),
            pltpu.VMEM((CHUNK, FEAT_DIM), jnp.float32),
            pltpu.VMEM((ROWS_PER_W,), jnp.float32),
            pltpu.SemaphoreType.DMA,
            pltpu.SemaphoreType.DMA,
        ],
    )
    return run(embed, head, rel, tail, w_relations)


def kernel(embed, head, rel, tail, w_relations):
    head = head.astype(jnp.int32)
    rel = rel.astype(jnp.int32)
    tail = tail.astype(jnp.int32)
    return _dist_mult(embed, head, rel, tail, w_relations)


# speedup vs baseline: 2.5311x; 1.0130x over previous
import jax
import jax.numpy as jnp
from jax import lax
from jax.experimental import pallas as pl
from jax.experimental.pallas import tpu as pltpu
from jax.experimental.pallas import tpu_sc as plsc

NUM_ENTITIES = 100000
NUM_RELS = 1000
FEAT_DIM = 128
BATCH = 16384

NC = 2
NS = 16
L = 16
NW = NC * NS
ROWS_PER_W = BATCH // NW
CHUNK = 128
NCHUNKS = ROWS_PER_W // CHUNK
GROUPS = CHUNK // L
UNROLL = 16
NACC = 4


def _sc_body(embed_hbm, head_hbm, rel_hbm, tail_hbm, wrel_hbm, out_hbm,
             hidx_v, ridx_v, tidx_v,
             h0, r0, t0, h1, r1, t1, out_v, sem0, sem1):
    wid = lax.axis_index("s") * NC + lax.axis_index("c")
    base = wid * ROWS_PER_W

    pltpu.sync_copy(head_hbm.at[pl.ds(base, ROWS_PER_W)], hidx_v)
    pltpu.sync_copy(rel_hbm.at[pl.ds(base, ROWS_PER_W)], ridx_v)
    pltpu.sync_copy(tail_hbm.at[pl.ds(base, ROWS_PER_W)], tidx_v)

    bufs = [(h0, r0, t0, sem0), (h1, r1, t1, sem1)]
    lane = lax.broadcasted_iota(jnp.int32, (L,), 0)

    def issue(c):
        hb, rb, tb, sem = bufs[c % 2]
        sl = pl.ds(c * CHUNK, CHUNK)
        return [
            pltpu.async_copy(embed_hbm.at[hidx_v.at[sl]], hb, sem),
            pltpu.async_copy(wrel_hbm.at[ridx_v.at[sl]], rb, sem),
            pltpu.async_copy(embed_hbm.at[tidx_v.at[sl]], tb, sem),
        ]

    pending = {0: issue(0)}
    for c in range(NCHUNKS):
        if c + 1 < NCHUNKS:
            pending[c + 1] = issue(c + 1)
        for cp in pending.pop(c):
            cp.wait()
        h_v, r_v, t_v, _ = bufs[c % 2]

        def group_body(g, _):
            rows = g * L + lane

            def dstep(_, carry):
                accs, col = carry
                accs = list(accs)
                for u in range(UNROLL):
                    hv = plsc.load_gather(h_v, [rows, col])
                    rv = plsc.load_gather(r_v, [rows, col])
                    tv = plsc.load_gather(t_v, [rows, col])
                    accs[u % NACC] = accs[u % NACC] + hv * rv * tv
                    col = (col + 1) & (FEAT_DIM - 1)
                return tuple(accs), col

            zero = jnp.zeros((L,), jnp.float32)
            accs, _ = lax.fori_loop(
                0, FEAT_DIM // UNROLL, dstep,
                ((zero,) * NACC, lane))
            acc = (accs[0] + accs[1]) + (accs[2] + accs[3])
            out_v[pl.ds(c * CHUNK + g * L, L)] = acc
            return 0

        lax.fori_loop(0, GROUPS, group_body, 0)

    pltpu.sync_copy(out_v, out_hbm.at[pl.ds(base, ROWS_PER_W)])


@jax.jit
def _dist_mult(embed, head, rel, tail, w_relations):
    mesh = plsc.VectorSubcoreMesh(core_axis_name="c", subcore_axis_name="s")
    run = pl.kernel(
        _sc_body,
        out_type=jax.ShapeDtypeStruct((BATCH,), jnp.float32),
        mesh=mesh,
        compiler_params=pltpu.CompilerParams(needs_layout_passes=False),
        scratch_types=[
            pltpu.VMEM((ROWS_PER_W,), jnp.int32),
            pltpu.VMEM((ROWS_PER_W,), jnp.int32),
            pltpu.VMEM((ROWS_PER_W,), jnp.int32),
            pltpu.VMEM((CHUNK, FEAT_DIM), jnp.float32),
            pltpu.VMEM((CHUNK, FEAT_DIM), jnp.float32),
            pltpu.VMEM((CHUNK, FEAT_DIM), jnp.float32),
            pltpu.VMEM((CHUNK, FEAT_DIM), jnp.float32),
            pltpu.VMEM((CHUNK, FEAT_DIM), jnp.float32),
            pltpu.VMEM((CHUNK, FEAT_DIM), jnp.float32),
            pltpu.VMEM((ROWS_PER_W,), jnp.float32),
            pltpu.SemaphoreType.DMA,
            pltpu.SemaphoreType.DMA,
        ],
    )
    return run(embed, head, rel, tail, w_relations)


def kernel(embed, head, rel, tail, w_relations):
    head = head.astype(jnp.int32)
    rel = rel.astype(jnp.int32)
    tail = tail.astype(jnp.int32)
    return _dist_mult(embed, head, rel, tail, w_relations)


# speedup vs baseline: 2.5763x; 1.0178x over previous
import jax
import jax.numpy as jnp
from jax import lax
from jax.experimental import pallas as pl
from jax.experimental.pallas import tpu as pltpu
from jax.experimental.pallas import tpu_sc as plsc

NUM_ENTITIES = 100000
NUM_RELS = 1000
FEAT_DIM = 128
BATCH = 16384

NC = 2
NS = 16
L = 16
NW = NC * NS
ROWS_PER_W = BATCH // NW
CHUNK = 64
NCHUNKS = ROWS_PER_W // CHUNK
NBUF = 4
GROUPS = CHUNK // L
UNROLL = 16
NACC = 4


def _sc_body(embed_hbm, head_hbm, rel_hbm, tail_hbm, wrel_hbm, out_hbm,
             hidx_v, ridx_v, tidx_v, hbufs, rbufs, tbufs, out_v, sems):
    wid = lax.axis_index("s") * NC + lax.axis_index("c")
    base = wid * ROWS_PER_W

    cp_i = [
        pltpu.async_copy(head_hbm.at[pl.ds(base, ROWS_PER_W)], hidx_v, sems[0]),
        pltpu.async_copy(rel_hbm.at[pl.ds(base, ROWS_PER_W)], ridx_v, sems[1]),
        pltpu.async_copy(tail_hbm.at[pl.ds(base, ROWS_PER_W)], tidx_v, sems[2]),
    ]
    for cp in cp_i:
        cp.wait()

    lane = lax.broadcasted_iota(jnp.int32, (L,), 0)

    def issue(c):
        k = c % NBUF
        sl = pl.ds(c * CHUNK, CHUNK)
        return [
            pltpu.async_copy(embed_hbm.at[hidx_v.at[sl]], hbufs[k], sems[k]),
            pltpu.async_copy(wrel_hbm.at[ridx_v.at[sl]], rbufs[k], sems[k]),
            pltpu.async_copy(embed_hbm.at[tidx_v.at[sl]], tbufs[k], sems[k]),
        ]

    pending = {}
    for c in range(NBUF - 1):
        pending[c] = issue(c)

    for c in range(NCHUNKS):
        if c + NBUF - 1 < NCHUNKS:
            pending[c + NBUF - 1] = issue(c + NBUF - 1)
        for cp in pending.pop(c):
            cp.wait()
        k = c % NBUF
        h_v, r_v, t_v = hbufs[k], rbufs[k], tbufs[k]

        def group_body(g, _):
            rows = g * L + lane

            def dstep(_, carry):
                accs, col = carry
                accs = list(accs)
                for u in range(UNROLL):
                    hv = plsc.load_gather(h_v, [rows, col])
                    rv = plsc.load_gather(r_v, [rows, col])
                    tv = plsc.load_gather(t_v, [rows, col])
                    accs[u % NACC] = accs[u % NACC] + hv * rv * tv
                    col = (col + 1) & (FEAT_DIM - 1)
                return tuple(accs), col

            zero = jnp.zeros((L,), jnp.float32)
            accs, _ = lax.fori_loop(
                0, FEAT_DIM // UNROLL, dstep,
                ((zero,) * NACC, lane))
            acc = (accs[0] + accs[1]) + (accs[2] + accs[3])
            out_v[pl.ds(c * CHUNK + g * L, L)] = acc
            return 0

        lax.fori_loop(0, GROUPS, group_body, 0)

    pltpu.sync_copy(out_v, out_hbm.at[pl.ds(base, ROWS_PER_W)])


@jax.jit
def _dist_mult(embed, head, rel, tail, w_relations):
    mesh = plsc.VectorSubcoreMesh(core_axis_name="c", subcore_axis_name="s")
    rowbuf = pltpu.VMEM((CHUNK, FEAT_DIM), jnp.float32)
    run = pl.kernel(
        _sc_body,
        out_type=jax.ShapeDtypeStruct((BATCH,), jnp.float32),
        mesh=mesh,
        compiler_params=pltpu.CompilerParams(needs_layout_passes=False),
        scratch_types=[
            pltpu.VMEM((ROWS_PER_W,), jnp.int32),
            pltpu.VMEM((ROWS_PER_W,), jnp.int32),
            pltpu.VMEM((ROWS_PER_W,), jnp.int32),
            [rowbuf] * NBUF,
            [rowbuf] * NBUF,
            [rowbuf] * NBUF,
            pltpu.VMEM((ROWS_PER_W,), jnp.float32),
            [pltpu.SemaphoreType.DMA] * NBUF,
        ],
    )
    return run(embed, head, rel, tail, w_relations)


def kernel(embed, head, rel, tail, w_relations):
    head = head.astype(jnp.int32)
    rel = rel.astype(jnp.int32)
    tail = tail.astype(jnp.int32)
    return _dist_mult(embed, head, rel, tail, w_relations)
